# baseline (speedup 1.0000x reference)
import jax
import jax.numpy as jnp
from jax.experimental import pallas as pl
from jax.experimental.pallas import tpu as pltpu

_CHUNKS = (512, 1024, 2048, 4608, 4608, 2048, 1024, 512)
_OFFSETS = tuple(sum(_CHUNKS[:k]) for k in range(len(_CHUNKS)))
_NC = len(_CHUNKS)


def _copy_kernel(x_hbm, o_hbm, *rest):
    bufs, sems_in, sems_out = rest[:_NC], rest[_NC:2 * _NC], rest[2 * _NC:]
    cps_in = [
        pltpu.make_async_copy(
            x_hbm.at[:, pl.ds(_OFFSETS[k], _CHUNKS[k])], bufs[k], sems_in[k])
        for k in range(_NC)
    ]
    cps_out = [
        pltpu.make_async_copy(
            bufs[k], o_hbm.at[:, pl.ds(_OFFSETS[k], _CHUNKS[k])], sems_out[k])
        for k in range(_NC)
    ]
    for cp in cps_in:
        cp.start()
    for k in range(_NC):
        cps_in[k].wait()
        cps_out[k].start()
    for cp in cps_out:
        cp.wait()


def kernel(method_soft, label, features):
    del label, features
    n, d = method_soft.shape
    xt = method_soft.T
    yt = pl.pallas_call(
        _copy_kernel,
        out_shape=jax.ShapeDtypeStruct((d, n), method_soft.dtype),
        in_specs=[pl.BlockSpec(memory_space=pl.ANY)],
        out_specs=pl.BlockSpec(memory_space=pl.ANY),
        scratch_shapes=(
            [pltpu.VMEM((d, c), method_soft.dtype) for c in _CHUNKS]
            + [pltpu.SemaphoreType.DMA] * (2 * _NC)
        ),
    )(xt)
    return yt.T


# speedup vs baseline: 1.0190x; 1.0190x over previous
import jax
import jax.numpy as jnp
from jax.experimental import pallas as pl
from jax.experimental.pallas import tpu as pltpu

_CHUNKS = (1024, 2048, 4096, 4096, 4096, 1024)
_OFFSETS = tuple(sum(_CHUNKS[:k]) for k in range(len(_CHUNKS)))
_NC = len(_CHUNKS)


def _copy_kernel(x_hbm, o_hbm, *rest):
    bufs, sems_in, sems_out = rest[:_NC], rest[_NC:2 * _NC], rest[2 * _NC:]
    cps_in = [
        pltpu.make_async_copy(
            x_hbm.at[:, pl.ds(_OFFSETS[k], _CHUNKS[k])], bufs[k], sems_in[k])
        for k in range(_NC)
    ]
    cps_out = [
        pltpu.make_async_copy(
            bufs[k], o_hbm.at[:, pl.ds(_OFFSETS[k], _CHUNKS[k])], sems_out[k])
        for k in range(_NC)
    ]
    for cp in cps_in:
        cp.start()
    for k in range(_NC):
        cps_in[k].wait()
        cps_out[k].start()
    for cp in cps_out:
        cp.wait()


def kernel(method_soft, label, features):
    del label, features
    n, d = method_soft.shape
    xt = method_soft.T
    yt = pl.pallas_call(
        _copy_kernel,
        out_shape=jax.ShapeDtypeStruct((d, n), method_soft.dtype),
        in_specs=[pl.BlockSpec(memory_space=pl.ANY)],
        out_specs=pl.BlockSpec(memory_space=pl.ANY),
        scratch_shapes=(
            [pltpu.VMEM((d, c), method_soft.dtype) for c in _CHUNKS]
            + [pltpu.SemaphoreType.DMA] * (2 * _NC)
        ),
    )(xt)
    return yt.T
